# baseline (speedup 1.0000x reference)
import functools

import jax
import jax.numpy as jnp
from jax import lax
from jax.experimental import pallas as pl
from jax.experimental.pallas import tpu as pltpu
from jax.experimental.pallas import tpu_sc as plsc

BATCH = 4
SEQ = 2048
HIDDEN = 1024
L = 16
NW = 32
CHUNK = SEQ // (NW // BATCH)
G = 16
N_SUB = CHUNK // G
DEPTH = 3
TR = DEPTH + 1
PR = DEPTH


def _bcast_lane(v, j):
    idx = jnp.broadcast_to(jnp.int32(j), (L, 1))
    dnums = lax.GatherDimensionNumbers(
        offset_dims=(), collapsed_slice_dims=(0,), start_index_map=(0,))
    return lax.gather(v, idx, dnums, (1,),
                      mode=lax.GatherScatterMode.PROMISE_IN_BOUNDS)


def _body(ids_hbm, mask_hbm, token_hbm, pos_hbm, out_hbm,
          ids_v, mask_v, pos_v, row0, *rest):
    tok_bufs = rest[:TR]
    pos_bufs = rest[TR:TR + PR]
    gsem = rest[TR + PR:2 * TR + PR]
    psem = rest[2 * TR + PR:2 * TR + 2 * PR]
    osem = rest[2 * TR + 2 * PR:3 * TR + 2 * PR]

    cid = lax.axis_index("c")
    sid = lax.axis_index("s")
    wid = sid * 2 + cid
    b = wid // 8
    c = wid % 8
    s0 = c * CHUNK

    def fire_tok(g):
        return pltpu.async_copy(
            token_hbm.at[ids_v.at[pl.ds(g * G, G)]],
            tok_bufs[g % TR], gsem[g % TR])

    def fire_pos(g):
        return pltpu.async_copy(
            pos_hbm.at[pos_v.at[pl.ds(g * G, G)]],
            pos_bufs[g % PR], psem[g % PR])

    pltpu.sync_copy(ids_hbm.at[b, pl.ds(s0, CHUNK)], ids_v)
    pltpu.sync_copy(mask_hbm.at[b, pl.ds(0, SEQ)], mask_v)
    pltpu.sync_copy(pos_hbm.at[pl.ds(0, 1)], row0)
    tok_cps = {g: fire_tok(g) for g in range(DEPTH)}

    def pstep(i, acc):
        return acc + mask_v[pl.ds(i * L, L)]
    acc = lax.fori_loop(0, s0 // L, pstep, jnp.zeros((L,), jnp.int32))
    prefix = jnp.sum(acc)

    def cstep(i, carry):
        m = mask_v[pl.ds(s0 + i * L, L)]
        cs = plsc.cumsum(m)
        dummy = SEQ + s0 + i * L + lax.iota(jnp.int32, L)
        pos = jnp.where(m == 0, dummy, carry + cs - 1)
        pos_v[pl.ds(i * L, L)] = pos
        return carry + jnp.sum(m)
    lax.fori_loop(0, CHUNK // L, cstep, prefix)

    pos_cps = {g: fire_pos(g) for g in range(DEPTH)}

    def add_block(g, tb, pb):
        mfv = (mask_v[pl.ds(s0 + g * G, G)]).astype(jnp.float32)

        def col_step(k, _):
            off = pl.multiple_of(k * L, L)
            r0 = row0[0, pl.ds(off, L)]
            for j in range(G):
                mf = _bcast_lane(mfv, j)
                plsc.addupdate(
                    tb.at[j, pl.ds(off, L)],
                    (pb[j, pl.ds(off, L)] - r0) * mf + r0)
            return 0
        lax.fori_loop(0, HIDDEN // L, col_step, 0)

    stores = {}
    for g in range(N_SUB):
        tok_cps.pop(g).wait()
        pos_cps.pop(g).wait()
        add_block(g, tok_bufs[g % TR], pos_bufs[g % PR])
        stores[g] = pltpu.async_copy(
            tok_bufs[g % TR],
            out_hbm.at[pl.ds(s0 + g * G, G), b],
            osem[g % TR])
        nxt = g + DEPTH
        if nxt < N_SUB:
            if nxt - TR in stores:
                stores.pop(nxt - TR).wait()
            tok_cps[nxt] = fire_tok(nxt)
            pos_cps[nxt] = fire_pos(nxt)
    for g in sorted(stores):
        stores.pop(g).wait()


@jax.jit
def _embed(input_ids, input_mask, token_table, pos_table):
    mesh = plsc.VectorSubcoreMesh(core_axis_name="c", subcore_axis_name="s")
    k = functools.partial(
        pl.kernel,
        mesh=mesh,
        compiler_params=pltpu.CompilerParams(needs_layout_passes=False),
        out_type=jax.ShapeDtypeStruct((SEQ, BATCH, HIDDEN), jnp.float32),
        scratch_types=[
            pltpu.VMEM((CHUNK,), jnp.int32),
            pltpu.VMEM((SEQ,), jnp.int32),
            pltpu.VMEM((CHUNK,), jnp.int32),
            pltpu.VMEM((1, HIDDEN), jnp.float32),
            *[pltpu.VMEM((G, HIDDEN), jnp.float32) for _ in range(TR + PR)],
            *[pltpu.SemaphoreType.DMA for _ in range(2 * TR + 2 * PR)],
        ],
    )(_body)
    return k(input_ids, input_mask, token_table, pos_table)


def kernel(input_ids, input_mask, token_table, pos_table):
    return _embed(input_ids, input_mask.astype(jnp.int32), token_table,
                  pos_table)


# speedup vs baseline: 1.0228x; 1.0228x over previous
import functools

import jax
import jax.numpy as jnp
from jax import lax
from jax.experimental import pallas as pl
from jax.experimental.pallas import tpu as pltpu
from jax.experimental.pallas import tpu_sc as plsc

BATCH = 4
SEQ = 2048
HIDDEN = 1024
L = 16
NW = 32
CHUNK = SEQ // (NW // BATCH)
G = 16
N_SUB = CHUNK // G
DEPTH = 2
TR = DEPTH + 1
PR = DEPTH


def _bcast_lane(v, j):
    idx = jnp.broadcast_to(jnp.int32(j), (L, 1))
    dnums = lax.GatherDimensionNumbers(
        offset_dims=(), collapsed_slice_dims=(0,), start_index_map=(0,))
    return lax.gather(v, idx, dnums, (1,),
                      mode=lax.GatherScatterMode.PROMISE_IN_BOUNDS)


def _body(ids_hbm, mask_hbm, token_hbm, pos_hbm, out_hbm,
          ids_v, mask_v, pos_v, row0, *rest):
    tok_bufs = rest[:TR]
    pos_bufs = rest[TR:TR + PR]
    gsem = rest[TR + PR:2 * TR + PR]
    psem = rest[2 * TR + PR:2 * TR + 2 * PR]
    osem = rest[2 * TR + 2 * PR:3 * TR + 2 * PR]

    cid = lax.axis_index("c")
    sid = lax.axis_index("s")
    wid = sid * 2 + cid
    b = wid // 8
    c = wid % 8
    s0 = c * CHUNK

    def fire_tok(g):
        return pltpu.async_copy(
            token_hbm.at[ids_v.at[pl.ds(g * G, G)]],
            tok_bufs[g % TR], gsem[g % TR])

    def fire_pos(g):
        return pltpu.async_copy(
            pos_hbm.at[pos_v.at[pl.ds(g * G, G)]],
            pos_bufs[g % PR], psem[g % PR])

    pltpu.sync_copy(ids_hbm.at[b, pl.ds(s0, CHUNK)], ids_v)
    tok_cps = {g: fire_tok(g) for g in range(DEPTH)}
    pltpu.sync_copy(mask_hbm.at[b, pl.ds(0, SEQ)], mask_v)
    pltpu.sync_copy(pos_hbm.at[pl.ds(0, 1)], row0)

    def pstep(i, acc):
        return acc + mask_v[pl.ds(i * L, L)]
    acc = lax.fori_loop(0, s0 // L, pstep, jnp.zeros((L,), jnp.int32))
    prefix = jnp.sum(acc)

    def cstep(i, carry):
        m = mask_v[pl.ds(s0 + i * L, L)]
        cs = plsc.cumsum(m)
        dummy = SEQ + s0 + i * L + lax.iota(jnp.int32, L)
        pos = jnp.where(m == 0, dummy, carry + cs - 1)
        pos_v[pl.ds(i * L, L)] = pos
        return carry + jnp.sum(m)
    lax.fori_loop(0, CHUNK // L, cstep, prefix)

    pos_cps = {g: fire_pos(g) for g in range(DEPTH)}

    def add_block(g, tb, pb):
        mfv = (mask_v[pl.ds(s0 + g * G, G)]).astype(jnp.float32)

        def col_step(k, _):
            off = pl.multiple_of(k * L, L)
            r0 = row0[0, pl.ds(off, L)]
            for j in range(G):
                mf = _bcast_lane(mfv, j)
                plsc.addupdate(
                    tb.at[j, pl.ds(off, L)],
                    (pb[j, pl.ds(off, L)] - r0) * mf + r0)
            return 0
        lax.fori_loop(0, HIDDEN // L, col_step, 0)

    stores = {}
    for g in range(N_SUB):
        tok_cps.pop(g).wait()
        pos_cps.pop(g).wait()
        add_block(g, tok_bufs[g % TR], pos_bufs[g % PR])
        stores[g] = pltpu.async_copy(
            tok_bufs[g % TR],
            out_hbm.at[pl.ds(s0 + g * G, G), b],
            osem[g % TR])
        nxt = g + DEPTH
        if nxt < N_SUB:
            if nxt - TR in stores:
                stores.pop(nxt - TR).wait()
            tok_cps[nxt] = fire_tok(nxt)
            pos_cps[nxt] = fire_pos(nxt)
    for g in sorted(stores):
        stores.pop(g).wait()


@jax.jit
def _embed(input_ids, input_mask, token_table, pos_table):
    mesh = plsc.VectorSubcoreMesh(core_axis_name="c", subcore_axis_name="s")
    k = functools.partial(
        pl.kernel,
        mesh=mesh,
        compiler_params=pltpu.CompilerParams(needs_layout_passes=False),
        out_type=jax.ShapeDtypeStruct((SEQ, BATCH, HIDDEN), jnp.float32),
        scratch_types=[
            pltpu.VMEM((CHUNK,), jnp.int32),
            pltpu.VMEM((SEQ,), jnp.int32),
            pltpu.VMEM((CHUNK,), jnp.int32),
            pltpu.VMEM((1, HIDDEN), jnp.float32),
            *[pltpu.VMEM((G, HIDDEN), jnp.float32) for _ in range(TR + PR)],
            *[pltpu.SemaphoreType.DMA for _ in range(2 * TR + 2 * PR)],
        ],
    )(_body)
    return k(input_ids, input_mask, token_table, pos_table)


def kernel(input_ids, input_mask, token_table, pos_table):
    return _embed(input_ids, input_mask.astype(jnp.int32), token_table,
                  pos_table)
